# baseline (speedup 1.0000x reference)
import functools

import jax
import jax.numpy as jnp
from jax import lax
from jax.experimental import pallas as pl
from jax.experimental.pallas import tpu as pltpu
from jax.experimental.pallas import tpu_sc as plsc

_NBUF = 8


def _round_up(x, m):
    return (x + m - 1) // m * m


def _pack_table(weight):
    V, D = weight.shape
    wb = weight.astype(jnp.bfloat16)
    return lax.bitcast_convert_type(wb.reshape(V, D // 2, 2), jnp.int32)


def _make_kernel(B, S, D, C):
    try:
        info = plsc.get_sparse_core_info()
        NC, NS, L = info.num_cores, info.num_subcores, info.num_lanes
    except ValueError:
        NC, NS, L = 2, 16, 16
    NW = NC * NS
    assert B % NW == 0
    BW = B // NW
    assert D % (2 * L) == 0
    DP = D // 2
    UH = DP // L
    SP = _round_up(S, 8)
    assert BW % _NBUF == 0

    mesh = plsc.VectorSubcoreMesh(core_axis_name="c", subcore_axis_name="s",
                                  num_cores=NC, num_subcores=NS)

    @functools.partial(
        pl.kernel,
        out_type=jax.ShapeDtypeStruct((B, D), jnp.float32),
        mesh=mesh,
        compiler_params=pltpu.CompilerParams(needs_layout_passes=False,
                                             use_tc_tiling_on_sc=False),
        scratch_types=[
            pltpu.VMEM((BW, S), jnp.int32),
            pltpu.VMEM((BW * SP,), jnp.int32),
            [pltpu.VMEM((S, DP), jnp.int32) for _ in range(_NBUF)],
            pltpu.VMEM((BW, D), jnp.float32),
            pltpu.VMEM((D,), jnp.float32),
            [pltpu.SemaphoreType.DMA for _ in range(_NBUF)],
        ],
    )
    def k(idx_hbm, w_hbm, bias_hbm, out_hbm,
          raw_v, ids_v, rows, acc_v, bias_v, sems):
        wid = lax.axis_index("s") * NC + lax.axis_index("c")
        base = wid * BW
        pltpu.sync_copy(idx_hbm.at[pl.ds(base, BW)], raw_v)
        pltpu.sync_copy(bias_hbm, bias_v)

        lane = lax.iota(jnp.int32, L)

        starts = [v * L for v in range(S // L)]
        if S % L:
            starts.append(S - L)

        def tok_body(j, carry):
            for p0 in starts:
                tok = raw_v[j, pl.ds(p0, L)] + (lane + p0) * C
                ids_v[pl.ds(j * SP + p0, L)] = tok
            return carry

        lax.fori_loop(0, _NBUF, tok_body, 0)

        def fire(j, buf, sem):
            pltpu.async_copy(w_hbm.at[ids_v.at[pl.ds(j * SP, S)]], buf, sem)

        def wait(buf, sem):
            pltpu.make_async_copy(w_hbm.at[ids_v.at[pl.ds(0, S)]], buf, sem).wait()

        bias_regs = tuple(bias_v[pl.ds(g * L, L)] for g in range(2 * UH))

        def accum(j, buf):
            def body(r, accs):
                words = [buf[r, pl.ds(u * L, L)] for u in range(UH)]
                new = list(accs)
                for u, w in enumerate(words):
                    new[2 * u] = new[2 * u] + lax.bitcast_convert_type(
                        w << 16, jnp.float32)
                for u, w in enumerate(words):
                    new[2 * u + 1] = new[2 * u + 1] + lax.bitcast_convert_type(
                        w, jnp.float32)
                return tuple(new)

            accs = lax.fori_loop(0, S, body, bias_regs, unroll=2)
            for g in range(2 * UH):
                acc_v[j, pl.ds(g * L, L)] = accs[g]

        for b in range(_NBUF):
            fire(b, rows[b], sems[b])

        lax.fori_loop(_NBUF, BW, tok_body, 0)

        def bag_body(t, carry):
            for b in range(_NBUF):
                j = _NBUF * t + b
                wait(rows[b], sems[b])
                accum(j, rows[b])

                @pl.when(j + _NBUF < BW)
                def _():
                    fire(j + _NBUF, rows[b], sems[b])

            return carry

        lax.fori_loop(0, BW // _NBUF, bag_body, 0)

        pltpu.sync_copy(acc_v, out_hbm.at[pl.ds(base, BW)])

    return k


def kernel(indices, weight, bias):
    B, S = indices.shape
    V, D = weight.shape
    C = V // S
    k = _make_kernel(B, S, D, C)
    G = D // 32
    bias_p = bias.reshape(G, 16, 2).transpose(0, 2, 1).reshape(D)
    out = k(indices.astype(jnp.int32), _pack_table(weight), bias_p)
    return out.reshape(B, G, 2, 16).transpose(0, 1, 3, 2).reshape(B, D)


# speedup vs baseline: 3.5638x; 3.5638x over previous
import functools

import jax
import jax.numpy as jnp
from jax import lax
from jax.experimental import pallas as pl
from jax.experimental.pallas import tpu as pltpu
from jax.experimental.pallas import tpu_sc as plsc

_NBUF = 8
_PACK_ROWS = 2000


def _round_up(x, m):
    return (x + m - 1) // m * m


def _pack_kernel(w_ref, out_ref):
    x = lax.bitcast_convert_type(w_ref[...], jnp.int32)
    rn = x + 0x7FFF + ((x >> 16) & 1)
    D = x.shape[1]
    lo = (rn[:, : D // 2] >> 16) & 0xFFFF
    hi = rn[:, D // 2:] & jnp.int32(-65536)
    out_ref[...] = lo | hi


def _pack_table(weight):
    V, D = weight.shape
    R = _PACK_ROWS
    assert V % R == 0
    return pl.pallas_call(
        _pack_kernel,
        grid=(V // R,),
        in_specs=[pl.BlockSpec((R, D), lambda i: (i, 0))],
        out_specs=pl.BlockSpec((R, D // 2), lambda i: (i, 0)),
        out_shape=jax.ShapeDtypeStruct((V, D // 2), jnp.int32),
    )(weight)


def _make_kernel(B, S, D, C):
    try:
        info = plsc.get_sparse_core_info()
        NC, NS, L = info.num_cores, info.num_subcores, info.num_lanes
    except ValueError:
        NC, NS, L = 2, 16, 16
    NW = NC * NS
    assert B % NW == 0
    BW = B // NW
    assert D % (2 * L) == 0
    DP = D // 2
    UH = DP // L
    SP = _round_up(S, 8)
    assert BW % _NBUF == 0

    mesh = plsc.VectorSubcoreMesh(core_axis_name="c", subcore_axis_name="s",
                                  num_cores=NC, num_subcores=NS)

    @functools.partial(
        pl.kernel,
        out_type=jax.ShapeDtypeStruct((B, D), jnp.float32),
        mesh=mesh,
        compiler_params=pltpu.CompilerParams(needs_layout_passes=False,
                                             use_tc_tiling_on_sc=False),
        scratch_types=[
            pltpu.VMEM((BW, S), jnp.int32),
            pltpu.VMEM((BW * SP,), jnp.int32),
            [pltpu.VMEM((S, DP), jnp.int32) for _ in range(_NBUF)],
            pltpu.VMEM((BW, D), jnp.float32),
            pltpu.VMEM((D,), jnp.float32),
            [pltpu.SemaphoreType.DMA for _ in range(_NBUF)],
        ],
    )
    def k(idx_hbm, w_hbm, bias_hbm, out_hbm,
          raw_v, ids_v, rows, acc_v, bias_v, sems):
        wid = lax.axis_index("s") * NC + lax.axis_index("c")
        base = wid * BW
        pltpu.sync_copy(idx_hbm.at[pl.ds(base, BW)], raw_v)
        pltpu.sync_copy(bias_hbm, bias_v)

        lane = lax.iota(jnp.int32, L)

        starts = [v * L for v in range(S // L)]
        if S % L:
            starts.append(S - L)

        def tok_body(j, carry):
            for p0 in starts:
                tok = raw_v[j, pl.ds(p0, L)] + (lane + p0) * C
                ids_v[pl.ds(j * SP + p0, L)] = tok
            return carry

        lax.fori_loop(0, _NBUF, tok_body, 0)

        def fire(j, buf, sem):
            pltpu.async_copy(w_hbm.at[ids_v.at[pl.ds(j * SP, S)]], buf, sem)

        def wait(buf, sem):
            pltpu.make_async_copy(w_hbm.at[ids_v.at[pl.ds(0, S)]], buf, sem).wait()

        bias_regs = tuple(bias_v[pl.ds(u * L, L)] for u in range(2 * UH))

        def accum(j, buf):
            def body(r, accs):
                words = [buf[r, pl.ds(u * L, L)] for u in range(UH)]
                new = list(accs)
                for u, w in enumerate(words):
                    new[u] = new[u] + lax.bitcast_convert_type(
                        w << 16, jnp.float32)
                for u, w in enumerate(words):
                    new[UH + u] = new[UH + u] + lax.bitcast_convert_type(
                        w, jnp.float32)
                return tuple(new)

            accs = lax.fori_loop(0, S, body, bias_regs, unroll=2)
            for g in range(2 * UH):
                acc_v[j, pl.ds(g * L, L)] = accs[g]

        for b in range(_NBUF):
            fire(b, rows[b], sems[b])

        lax.fori_loop(_NBUF, BW, tok_body, 0)

        def bag_body(t, carry):
            for b in range(_NBUF):
                j = _NBUF * t + b
                wait(rows[b], sems[b])
                accum(j, rows[b])

                @pl.when(j + _NBUF < BW)
                def _():
                    fire(j + _NBUF, rows[b], sems[b])

            return carry

        lax.fori_loop(0, BW // _NBUF, bag_body, 0)

        pltpu.sync_copy(acc_v, out_hbm.at[pl.ds(base, BW)])

    return k


def kernel(indices, weight, bias):
    B, S = indices.shape
    V, D = weight.shape
    C = V // S
    k = _make_kernel(B, S, D, C)
    return k(indices.astype(jnp.int32), _pack_table(weight), bias)


# speedup vs baseline: 3.9884x; 1.1191x over previous
import functools

import jax
import jax.numpy as jnp
from jax import lax
from jax.experimental import pallas as pl
from jax.experimental.pallas import tpu as pltpu
from jax.experimental.pallas import tpu_sc as plsc

_NBUF = 8
_PACK_ROWS = 4000


def _round_up(x, m):
    return (x + m - 1) // m * m


def _pack_kernel(w_ref, out_ref):
    x = lax.bitcast_convert_type(w_ref[...], jnp.uint32)
    rn = x + jnp.uint32(0x8000)
    D = x.shape[1]
    lo = rn[:, : D // 2] >> 16
    hi = rn[:, D // 2:] & jnp.uint32(0xFFFF0000)
    out_ref[...] = lax.bitcast_convert_type(lo | hi, jnp.int32)


def _pack_table(weight):
    V, D = weight.shape
    R = _PACK_ROWS
    assert V % R == 0
    return pl.pallas_call(
        _pack_kernel,
        grid=(V // R,),
        in_specs=[pl.BlockSpec((R, D), lambda i: (i, 0))],
        out_specs=pl.BlockSpec((R, D // 2), lambda i: (i, 0)),
        out_shape=jax.ShapeDtypeStruct((V, D // 2), jnp.int32),
    )(weight)


def _make_kernel(B, S, D, C):
    try:
        info = plsc.get_sparse_core_info()
        NC, NS, L = info.num_cores, info.num_subcores, info.num_lanes
    except ValueError:
        NC, NS, L = 2, 16, 16
    NW = NC * NS
    assert B % NW == 0
    BW = B // NW
    assert D % (2 * L) == 0
    DP = D // 2
    UH = DP // L
    SP = _round_up(S, 8)
    assert BW % _NBUF == 0

    mesh = plsc.VectorSubcoreMesh(core_axis_name="c", subcore_axis_name="s",
                                  num_cores=NC, num_subcores=NS)

    @functools.partial(
        pl.kernel,
        out_type=jax.ShapeDtypeStruct((B, D), jnp.float32),
        mesh=mesh,
        compiler_params=pltpu.CompilerParams(needs_layout_passes=False,
                                             use_tc_tiling_on_sc=False),
        scratch_types=[
            pltpu.VMEM((BW, S), jnp.int32),
            pltpu.VMEM((BW * SP,), jnp.int32),
            [pltpu.VMEM((S, DP), jnp.int32) for _ in range(_NBUF)],
            pltpu.VMEM((BW, D), jnp.float32),
            pltpu.VMEM((D,), jnp.float32),
            [pltpu.SemaphoreType.DMA for _ in range(_NBUF)],
        ],
    )
    def k(idx_hbm, w_hbm, bias_hbm, out_hbm,
          raw_v, ids_v, rows, acc_v, bias_v, sems):
        wid = lax.axis_index("s") * NC + lax.axis_index("c")
        base = wid * BW
        pltpu.sync_copy(idx_hbm.at[pl.ds(base, BW)], raw_v)
        pltpu.sync_copy(bias_hbm, bias_v)

        lane = lax.iota(jnp.int32, L)

        starts = [v * L for v in range(S // L)]
        if S % L:
            starts.append(S - L)

        def tok_body(j, carry):
            for p0 in starts:
                tok = raw_v[j, pl.ds(p0, L)] + (lane + p0) * C
                ids_v[pl.ds(j * SP + p0, L)] = tok
            return carry

        lax.fori_loop(0, _NBUF, tok_body, 0)

        def fire(j, buf, sem):
            pltpu.async_copy(w_hbm.at[ids_v.at[pl.ds(j * SP, S)]], buf, sem)

        def wait(buf, sem):
            pltpu.make_async_copy(w_hbm.at[ids_v.at[pl.ds(0, S)]], buf, sem).wait()

        bias_regs = tuple(bias_v[pl.ds(u * L, L)] for u in range(2 * UH))

        def accum(j, buf):
            def body(r, accs):
                words = [buf[r, pl.ds(u * L, L)] for u in range(UH)]
                new = list(accs)
                for u, w in enumerate(words):
                    new[u] = new[u] + lax.bitcast_convert_type(
                        w << 16, jnp.float32)
                for u, w in enumerate(words):
                    new[UH + u] = new[UH + u] + lax.bitcast_convert_type(
                        w, jnp.float32)
                return tuple(new)

            accs = lax.fori_loop(0, S, body, bias_regs, unroll=2)
            for g in range(2 * UH):
                acc_v[j, pl.ds(g * L, L)] = accs[g]

        for b in range(_NBUF):
            fire(b, rows[b], sems[b])

        lax.fori_loop(_NBUF, BW, tok_body, 0)

        def bag_body(t, carry):
            for b in range(_NBUF):
                j = _NBUF * t + b
                wait(rows[b], sems[b])
                accum(j, rows[b])

                @pl.when(j + _NBUF < BW)
                def _():
                    fire(j + _NBUF, rows[b], sems[b])

            return carry

        lax.fori_loop(0, BW // _NBUF, bag_body, 0)

        pltpu.sync_copy(acc_v, out_hbm.at[pl.ds(base, BW)])

    return k


def kernel(indices, weight, bias):
    B, S = indices.shape
    V, D = weight.shape
    C = V // S
    k = _make_kernel(B, S, D, C)
    return k(indices.astype(jnp.int32), _pack_table(weight), bias)


# speedup vs baseline: 4.8754x; 1.2224x over previous
import functools

import jax
import jax.numpy as jnp
from jax import lax
from jax.experimental import pallas as pl
from jax.experimental.pallas import tpu as pltpu
from jax.experimental.pallas import tpu_sc as plsc

_NBUF = 8
_PACK_ROWS = 4000


def _round_up(x, m):
    return (x + m - 1) // m * m


def _pack_kernel(w_ref, out_ref):
    x = lax.bitcast_convert_type(w_ref[...], jnp.uint32)
    rn = x + jnp.uint32(0x8000)
    R, D = x.shape
    lo = rn[:, : D // 2] >> 16
    hi = rn[:, D // 2:] & jnp.uint32(0xFFFF0000)
    packed = lax.bitcast_convert_type(lo | hi, jnp.int32)
    p3 = packed.reshape(R // 2, 2, D // 2)
    out_ref[...] = lax.concatenate([p3[:, 0, :], p3[:, 1, :]], 1)


def _pack_table(weight):
    V, D = weight.shape
    R = _PACK_ROWS
    assert V % R == 0
    return pl.pallas_call(
        _pack_kernel,
        grid=(V // R,),
        in_specs=[pl.BlockSpec((R, D), lambda i: (i, 0))],
        out_specs=pl.BlockSpec((R // 2, D), lambda i: (i, 0)),
        out_shape=jax.ShapeDtypeStruct((V // 2, D), jnp.int32),
    )(weight)


def _make_kernel(B, S, D, C):
    try:
        info = plsc.get_sparse_core_info()
        NC, NS, L = info.num_cores, info.num_subcores, info.num_lanes
    except ValueError:
        NC, NS, L = 2, 16, 16
    NW = NC * NS
    assert B % NW == 0
    BW = B // NW
    assert D % (2 * L) == 0
    DP = D // 2
    UH = DP // L
    SP = _round_up(S, 8)
    assert BW % _NBUF == 0

    mesh = plsc.VectorSubcoreMesh(core_axis_name="c", subcore_axis_name="s",
                                  num_cores=NC, num_subcores=NS)

    @functools.partial(
        pl.kernel,
        out_type=jax.ShapeDtypeStruct((B, D), jnp.float32),
        mesh=mesh,
        compiler_params=pltpu.CompilerParams(needs_layout_passes=False,
                                             use_tc_tiling_on_sc=False),
        scratch_types=[
            pltpu.VMEM((BW, S), jnp.int32),
            pltpu.VMEM((BW * SP,), jnp.int32),
            [pltpu.VMEM((S, DP), jnp.int32) for _ in range(_NBUF)],
            pltpu.VMEM((BW, D), jnp.float32),
            pltpu.VMEM((D,), jnp.float32),
            [pltpu.SemaphoreType.DMA for _ in range(_NBUF)],
        ],
    )
    def k(idx_hbm, w_hbm, bias_hbm, out_hbm,
          raw_v, ids_v, rows, acc_v, bias_v, sems):
        wid = lax.axis_index("s") * NC + lax.axis_index("c")
        base = wid * BW
        pltpu.sync_copy(idx_hbm.at[pl.ds(base, BW)], raw_v)
        pltpu.sync_copy(bias_hbm, bias_v)

        lane = lax.iota(jnp.int32, L)

        starts = [v * L for v in range(S // L)]
        if S % L:
            starts.append(S - L)

        def tok_body(j, carry):
            for p0 in starts:
                tok = raw_v[j, pl.ds(p0, L)] + (lane + p0) * C
                ids_v[pl.ds(j * SP + p0, L)] = tok
            return carry

        lax.fori_loop(0, _NBUF, tok_body, 0)

        def fire(j, buf, sem):
            pltpu.async_copy(w_hbm.at[ids_v.at[pl.ds(j * SP, S)]], buf, sem)

        def wait(buf, sem):
            pltpu.make_async_copy(w_hbm.at[ids_v.at[pl.ds(0, S)]], buf, sem).wait()

        bias_regs = tuple(bias_v[pl.ds(u * L, L)] for u in range(2 * UH))

        def accum(j, buf):
            def body(r, accs):
                words = [buf[r, pl.ds(u * L, L)] for u in range(UH)]
                new = list(accs)
                for u, w in enumerate(words):
                    new[u] = new[u] + lax.bitcast_convert_type(
                        w << 16, jnp.float32)
                for u, w in enumerate(words):
                    new[UH + u] = new[UH + u] + lax.bitcast_convert_type(
                        w, jnp.float32)
                return tuple(new)

            accs = lax.fori_loop(0, S, body, bias_regs, unroll=2)
            for g in range(2 * UH):
                acc_v[j, pl.ds(g * L, L)] = accs[g]

        for b in range(_NBUF):
            fire(b, rows[b], sems[b])

        lax.fori_loop(_NBUF, BW, tok_body, 0)

        def bag_body(t, carry):
            for b in range(_NBUF):
                j = _NBUF * t + b
                wait(rows[b], sems[b])
                accum(j, rows[b])

                @pl.when(j + _NBUF < BW)
                def _():
                    fire(j + _NBUF, rows[b], sems[b])

            return carry

        lax.fori_loop(0, BW // _NBUF, bag_body, 0)

        pltpu.sync_copy(acc_v, out_hbm.at[pl.ds(base, BW)])

    return k


def kernel(indices, weight, bias):
    B, S = indices.shape
    V, D = weight.shape
    C = V // S
    k = _make_kernel(B, S, D, C)
    packed = _pack_table(weight).reshape(V, D // 2)
    return k(indices.astype(jnp.int32), packed, bias)
